# baseline (speedup 1.0000x reference)
import functools

import numpy as np

import jax
import jax.numpy as jnp
from jax import lax
from jax.experimental import pallas as pl
from jax.experimental.pallas import tpu as pltpu
from jax.experimental.pallas import tpu_sc as plsc

N, K, H, W = 4, 8, 256, 256
HW = H * W
C = 32
CP = C // 2
P = 100000
NC, NS, L = 2, 16, 16
NW = NC * NS
PPW = (N * HW) // NW
PARTS = NW // N
SUP = 1024
NSUP = PPW // SUP
CH = 128
SUBS = SUP // CH
IB = 128


def _sc_composite(table, frag, alpha):
    mesh = plsc.VectorSubcoreMesh(core_axis_name="c", subcore_axis_name="s")

    @functools.partial(
        pl.kernel,
        mesh=mesh,
        compiler_params=pltpu.CompilerParams(use_tc_tiling_on_sc=False,
                                             needs_layout_passes=False),
        out_type=jax.ShapeDtypeStruct((N, C, H, W), jnp.float32),
        scratch_types=[
            pltpu.VMEM((2, K, SUBS, IB), jnp.int32),
            pltpu.VMEM((2, K, SUP), jnp.float32),
            pltpu.VMEM((K, SUP), jnp.float32),
            pltpu.VMEM((2, K, CH, CP), jnp.int32),
            pltpu.VMEM((C, CH), jnp.float32),
            pltpu.SemaphoreType.DMA,
            pltpu.SemaphoreType.DMA,
        ],
    )
    def k(table_hbm, frag_hbm, alpha_hbm, out_hbm,
          idx_v, alpha_v, w_v, rows_v, acc_v, sem_g, sem_s):
        wid = lax.axis_index("s") * NC + lax.axis_index("c")
        n = wid // PARTS
        base_hw = (wid % PARTS) * PPW

        def frag_slice(hw):
            return frag_hbm.at[n, :, pl.ds(pl.multiple_of(hw // IB, 8),
                                           SUP // IB), :]

        def alpha_slice(hw):
            return alpha_hbm.at[n, :, pl.ds(hw, SUP)]

        def sup_hw(si):
            return pl.multiple_of(base_hw + si * SUP, SUP)

        pltpu.sync_copy(frag_slice(sup_hw(0)), idx_v.at[0])
        pltpu.sync_copy(alpha_slice(sup_hw(0)), alpha_v.at[0])
        for kk in range(K):
            pltpu.async_copy(table_hbm.at[idx_v.at[0, kk, 0]],
                             rows_v.at[0, kk], sem_g)

        def sup_body(si, _):
            b = si % 2
            hw0 = sup_hw(si)

            @pl.when(si + 1 < NSUP)
            def _():
                pltpu.async_copy(frag_slice(sup_hw(si + 1)),
                                 idx_v.at[1 - b], sem_s)
                pltpu.async_copy(alpha_slice(sup_hw(si + 1)),
                                 alpha_v.at[1 - b], sem_s)

            def wgrp(g, _):
                t = jnp.ones((L,), jnp.float32)
                for kk in range(K):
                    a = alpha_v[b, kk, pl.ds(g * L, L)]
                    w_v[kk, pl.ds(g * L, L)] = a * t
                    t = t * (1.0 - a)
                return 0
            lax.fori_loop(0, SUP // L, wgrp, 0)

            def sub_body(sj, _):
                rp = sj % 2
                np_ = (sj + 1) % 2

                for kk in range(K):
                    pltpu.make_async_copy(
                        table_hbm.at[idx_v.at[b, kk, sj]],
                        rows_v.at[rp, kk], sem_g).wait()

                @pl.when(sj < SUBS - 1)
                def _():
                    for kk in range(K):
                        pltpu.async_copy(
                            table_hbm.at[idx_v.at[b, kk, sj + 1]],
                            rows_v.at[np_, kk], sem_g)

                @pl.when(jnp.logical_and(sj == SUBS - 1, si < NSUP - 1))
                def _():
                    pltpu.make_async_copy(frag_slice(sup_hw(si + 1)),
                                          idx_v.at[1 - b], sem_s).wait()
                    pltpu.make_async_copy(alpha_slice(sup_hw(si + 1)),
                                          alpha_v.at[1 - b], sem_s).wait()
                    for kk in range(K):
                        pltpu.async_copy(
                            table_hbm.at[idx_v.at[1 - b, kk, 0]],
                            rows_v.at[np_, kk], sem_g)

                rows_lo = lax.iota(jnp.int32, L)
                rows_hi = rows_lo + L

                def px_body(g, _):
                    p0 = g * L
                    wvs = [w_v[kk, pl.ds(sj * CH + p0, L)] for kk in range(K)]
                    for i in range(L):
                        p = p0 + i
                        ri = rows_v[rp, 0, p, pl.ds(0, CP)]
                        w0 = wvs[0][i]
                        a0 = w0 * plsc.bitcast(ri << 16, jnp.float32)
                        a1 = w0 * plsc.bitcast(ri & -65536, jnp.float32)
                        for kk in range(1, K):
                            ri = rows_v[rp, kk, p, pl.ds(0, CP)]
                            wk = wvs[kk][i]
                            a0 = a0 + wk * plsc.bitcast(ri << 16, jnp.float32)
                            a1 = a1 + wk * plsc.bitcast(ri & -65536,
                                                        jnp.float32)
                        col = jnp.full((L,), p, jnp.int32)
                        plsc.store_scatter(acc_v, [rows_lo, col], a0)
                        plsc.store_scatter(acc_v, [rows_hi, col], a1)
                    return 0
                lax.fori_loop(0, CH // L, px_body, 0)

                hw = hw0 + sj * CH
                h_row = hw // W
                w0_ = pl.multiple_of(hw % W, CH)
                pltpu.sync_copy(acc_v,
                                out_hbm.at[n, :, h_row, pl.ds(w0_, CH)])
                return 0

            lax.fori_loop(0, SUBS, sub_body, 0)
            return 0

        lax.fori_loop(0, NSUP, sup_body, 0)

    return k(table, frag, alpha)


_PERM = np.stack([np.arange(CP), np.arange(CP, C)], axis=1).reshape(-1)


def kernel(fragments, alphas, ptclds):
    frag = fragments.astype(jnp.int32).reshape(N, K, HW // IB, IB)
    alpha = alphas.reshape(N, K, HW)
    tbl = ptclds.T[:, _PERM].astype(jnp.bfloat16)
    table = jax.lax.bitcast_convert_type(tbl.reshape(P, CP, 2), jnp.int32)
    return _sc_composite(table, frag, alpha)


# speedup vs baseline: 1.7099x; 1.7099x over previous
import functools

import jax
import jax.numpy as jnp
from jax import lax
from jax.experimental import pallas as pl
from jax.experimental.pallas import tpu as pltpu
from jax.experimental.pallas import tpu_sc as plsc

N, K, H, W = 4, 8, 256, 256
HW = H * W
C = 32
CP = C // 2
P = 100000
NC, NS, L = 2, 16, 16
NW = NC * NS
PPW = (N * HW) // NW
PARTS = NW // N
SUP = 1024
NSUP = PPW // SUP
CH = 128
SUBS = SUP // CH
IB = 128


def _sc_composite(table, frag, alpha):
    mesh = plsc.VectorSubcoreMesh(core_axis_name="c", subcore_axis_name="s")

    @functools.partial(
        pl.kernel,
        mesh=mesh,
        compiler_params=pltpu.CompilerParams(use_tc_tiling_on_sc=False,
                                             needs_layout_passes=False),
        out_type=jax.ShapeDtypeStruct((N, C, H, W), jnp.float32),
        scratch_types=[
            pltpu.VMEM((2, K, SUBS, IB), jnp.int32),
            pltpu.VMEM((2, K, SUP), jnp.float32),
            pltpu.VMEM((K, SUP), jnp.float32),
            pltpu.VMEM((2, K, CH, C), jnp.bfloat16),
            pltpu.VMEM((C, CH + 1), jnp.float32),
            pltpu.SemaphoreType.DMA,
            pltpu.SemaphoreType.DMA,
        ],
    )
    def k(table_hbm, frag_hbm, alpha_hbm, out_hbm,
          idx_v, alpha_v, w_v, rows_v, acc_v, sem_g, sem_s):
        wid = lax.axis_index("s") * NC + lax.axis_index("c")
        n = wid // PARTS
        base_hw = (wid % PARTS) * PPW

        def frag_slice(hw):
            return frag_hbm.at[n, :, pl.ds(pl.multiple_of(hw // IB, 8),
                                           SUP // IB), :]

        def alpha_slice(hw):
            return alpha_hbm.at[n, :, pl.ds(hw, SUP)]

        def sup_hw(si):
            return pl.multiple_of(base_hw + si * SUP, SUP)

        pltpu.sync_copy(frag_slice(sup_hw(0)), idx_v.at[0])
        pltpu.sync_copy(alpha_slice(sup_hw(0)), alpha_v.at[0])
        for kk in range(K):
            pltpu.async_copy(table_hbm.at[idx_v.at[0, kk, 0]],
                             rows_v.at[0, kk], sem_g)

        def sup_body(si, _):
            b = si % 2
            hw0 = sup_hw(si)

            @pl.when(si + 1 < NSUP)
            def _():
                pltpu.async_copy(frag_slice(sup_hw(si + 1)),
                                 idx_v.at[1 - b], sem_s)
                pltpu.async_copy(alpha_slice(sup_hw(si + 1)),
                                 alpha_v.at[1 - b], sem_s)

            def wgrp(g, _):
                t = jnp.ones((L,), jnp.float32)
                for kk in range(K):
                    a = alpha_v[b, kk, pl.ds(g * L, L)]
                    w_v[kk, pl.ds(g * L, L)] = a * t
                    t = t * (1.0 - a)
                return 0
            lax.fori_loop(0, SUP // L, wgrp, 0)

            def sub_body(sj, _):
                rp = sj % 2
                np_ = (sj + 1) % 2

                for kk in range(K):
                    pltpu.make_async_copy(
                        table_hbm.at[idx_v.at[b, kk, sj]],
                        rows_v.at[rp, kk], sem_g).wait()

                @pl.when(sj < SUBS - 1)
                def _():
                    for kk in range(K):
                        pltpu.async_copy(
                            table_hbm.at[idx_v.at[b, kk, sj + 1]],
                            rows_v.at[np_, kk], sem_g)

                @pl.when(jnp.logical_and(sj == SUBS - 1, si < NSUP - 1))
                def _():
                    pltpu.make_async_copy(frag_slice(sup_hw(si + 1)),
                                          idx_v.at[1 - b], sem_s).wait()
                    pltpu.make_async_copy(alpha_slice(sup_hw(si + 1)),
                                          alpha_v.at[1 - b], sem_s).wait()
                    for kk in range(K):
                        pltpu.async_copy(
                            table_hbm.at[idx_v.at[1 - b, kk, 0]],
                            rows_v.at[np_, kk], sem_g)

                rows_ev = 2 * lax.iota(jnp.int32, L)
                rows_od = rows_ev + 1

                def px_body(g, _):
                    p0 = g * L
                    wvs = [w_v[kk, pl.ds(sj * CH + p0, L)] for kk in range(K)]
                    for i in range(L):
                        p = p0 + i
                        ri = plsc.bitcast(rows_v[rp, 0, p, pl.ds(0, C)],
                                          jnp.int32)
                        w0 = wvs[0][i]
                        a0 = w0 * plsc.bitcast(ri << 16, jnp.float32)
                        a1 = w0 * plsc.bitcast(ri & -65536, jnp.float32)
                        for kk in range(1, K):
                            ri = plsc.bitcast(rows_v[rp, kk, p, pl.ds(0, C)],
                                              jnp.int32)
                            wk = wvs[kk][i]
                            a0 = a0 + wk * plsc.bitcast(ri << 16, jnp.float32)
                            a1 = a1 + wk * plsc.bitcast(ri & -65536,
                                                        jnp.float32)
                        col = jnp.full((L,), p, jnp.int32)
                        plsc.store_scatter(acc_v, [rows_ev, col], a0)
                        plsc.store_scatter(acc_v, [rows_od, col], a1)
                    return 0
                lax.fori_loop(0, CH // L, px_body, 0)

                hw = hw0 + sj * CH
                h_row = hw // W
                w0_ = pl.multiple_of(hw % W, CH)
                pltpu.sync_copy(acc_v.at[:, pl.ds(0, CH)],
                                out_hbm.at[n, :, h_row, pl.ds(w0_, CH)])
                return 0

            lax.fori_loop(0, SUBS, sub_body, 0)
            return 0

        lax.fori_loop(0, NSUP, sup_body, 0)

    return k(table, frag, alpha)


def kernel(fragments, alphas, ptclds):
    frag = fragments.astype(jnp.int32).reshape(N, K, HW // IB, IB)
    alpha = alphas.reshape(N, K, HW)
    table = ptclds.T.astype(jnp.bfloat16)
    return _sc_composite(table, frag, alpha)
